# Initial kernel scaffold; baseline (speedup 1.0000x reference)
import jax
import jax.numpy as jnp
from jax.experimental import pallas as pl


def kernel(in_feat, edge_index, W1, b1, W2, b2):
    raise NotImplementedError("write your pallas kernel here")


# speedup vs baseline: 4.8526x; 4.8526x over previous
import functools

import jax
import jax.numpy as jnp
from jax import lax
from jax.experimental import pallas as pl
from jax.experimental.pallas import tpu as pltpu
from jax.experimental.pallas import tpu_sc as plsc

N = 10000
NP = 10240
D_IN = 128
H1 = 512
H2 = 256
E = 320000
B = 128
NTILES = 32
EP = 323584
EPT1 = EP // 32
NB1 = EPT1 // B
EPT2 = EP // 16
NB2 = EPT2 // B
RPT = NP // 16

_mesh = plsc.VectorSubcoreMesh(core_axis_name="c", subcore_axis_name="s")
_f32 = jnp.float32


def _zero_vmem_block(zb, rows):
    z16 = jnp.zeros((16,), _f32)

    def body(i, carry):
        r = i // 8
        k = i % 8
        zb[r, pl.ds(k * 16, 16)] = z16
        return carry

    lax.fori_loop(0, rows * 8, body, 0)


@functools.partial(
    pl.kernel,
    mesh=_mesh,
    out_type=jax.ShapeDtypeStruct((64 * NP,), _f32),
    scratch_types=[
        pltpu.VMEM((NP,), _f32),
        pltpu.VMEM((NP,), _f32),
        pltpu.VMEM((B,), jnp.int32),
        pltpu.VMEM((B,), jnp.int32),
    ],
    compiler_params=pltpu.CompilerParams(needs_layout_passes=False),
)
def _deg_kernel(srcp, dstp, out, hs, hd, ib, jb):
    c = lax.axis_index("c")
    s = lax.axis_index("s")
    wid = s * 2 + c
    z16 = jnp.zeros((16,), _f32)

    def zero(i, carry):
        hs[pl.ds(i * 16, 16)] = z16
        hd[pl.ds(i * 16, 16)] = z16
        return carry

    lax.fori_loop(0, NP // 16, zero, 0)

    ones = jnp.ones((16,), _f32)
    base0 = wid * EPT1

    def batch(g, carry):
        base = base0 + g * B
        pltpu.sync_copy(srcp.at[pl.ds(base, B)], ib)
        pltpu.sync_copy(dstp.at[pl.ds(base, B)], jb)
        for j in range(B // 16):
            plsc.addupdate_scatter(hs, [ib[pl.ds(j * 16, 16)]], ones)
            plsc.addupdate_scatter(hd, [jb[pl.ds(j * 16, 16)]], ones)
        return carry

    lax.fori_loop(0, NB1, batch, 0)
    pltpu.sync_copy(hs, out.at[pl.ds(wid * NP, NP)])
    pltpu.sync_copy(hd, out.at[pl.ds((32 + wid) * NP, NP)])


@functools.partial(
    pl.kernel,
    mesh=_mesh,
    out_type=[
        jax.ShapeDtypeStruct((NP, D_IN), _f32),
        jax.ShapeDtypeStruct((NP, D_IN), _f32),
    ],
    scratch_types=[
        pltpu.VMEM_SHARED((NP, D_IN), _f32),
        pltpu.VMEM((B,), jnp.int32),
        pltpu.VMEM((B,), jnp.int32),
        pltpu.VMEM((B, D_IN), _f32),
        pltpu.VMEM((64, D_IN), _f32),
        pltpu.SemaphoreType.DMA,
    ],
)
def _mp1_kernel(xs, srcp, dstp, out0, out1, agg, ib, jb, rows, zb, sem):
    c = lax.axis_index("c")
    s = lax.axis_index("s")
    wid = s * 2 + c
    row0 = s * RPT

    _zero_vmem_block(zb, 64)

    def zcp(t, carry):
        pltpu.sync_copy(zb, agg.at[pl.ds(row0 + t * 64, 64)])
        return carry

    lax.fori_loop(0, RPT // 64, zcp, 0)
    plsc.subcore_barrier()

    base0 = wid * EPT1

    def batch(g, carry):
        base = base0 + g * B
        pltpu.sync_copy(srcp.at[pl.ds(base, B)], ib)
        pltpu.sync_copy(dstp.at[pl.ds(base, B)], jb)
        pltpu.async_copy(xs.at[ib], rows, sem).wait()
        pltpu.sync_copy(rows, agg.at[jb], add=True)
        return carry

    lax.fori_loop(0, NB1, batch, 0)
    plsc.subcore_barrier()

    @pl.when(c == 0)
    def _():
        pltpu.sync_copy(agg.at[pl.ds(row0, RPT)], out0.at[pl.ds(row0, RPT)])

    @pl.when(c == 1)
    def _():
        pltpu.sync_copy(agg.at[pl.ds(row0, RPT)], out1.at[pl.ds(row0, RPT)])


@functools.partial(
    pl.kernel,
    mesh=_mesh,
    out_type=[
        jax.ShapeDtypeStruct((NP, D_IN), _f32),
        jax.ShapeDtypeStruct((NP, D_IN), _f32),
    ],
    scratch_types=[
        pltpu.VMEM_SHARED((NP, D_IN), _f32),
        pltpu.VMEM((B,), jnp.int32),
        pltpu.VMEM((B,), jnp.int32),
        pltpu.VMEM((B, D_IN), _f32),
        pltpu.VMEM((64, D_IN), _f32),
        pltpu.SemaphoreType.DMA,
    ],
)
def _mp2_kernel(z0, z1, srcp, dstp, out0, out1, agg, ib, jb, rows, zb, sem):
    c = lax.axis_index("c")
    s = lax.axis_index("s")
    row0 = s * RPT

    _zero_vmem_block(zb, 64)

    def run(table, out):
        def zcp(t, carry):
            pltpu.sync_copy(zb, agg.at[pl.ds(row0 + t * 64, 64)])
            return carry

        lax.fori_loop(0, RPT // 64, zcp, 0)
        plsc.subcore_barrier()

        base0 = s * EPT2

        def batch(g, carry):
            base = base0 + g * B
            pltpu.sync_copy(srcp.at[pl.ds(base, B)], ib)
            pltpu.sync_copy(dstp.at[pl.ds(base, B)], jb)
            pltpu.async_copy(table.at[ib], rows, sem).wait()
            pltpu.sync_copy(rows, agg.at[jb], add=True)
            return carry

        lax.fori_loop(0, NB2, batch, 0)
        plsc.subcore_barrier()
        pltpu.sync_copy(agg.at[pl.ds(row0, RPT)], out.at[pl.ds(row0, RPT)])

    @pl.when(c == 0)
    def _():
        run(z0, out0)

    @pl.when(c == 1)
    def _():
        run(z1, out1)


_RB = 1024


def _prep_body(deg_ref, x_ref, xs_ref, on_ref, in_ref):
    dg = deg_ref[...]
    od = jnp.sum(dg[:32], axis=0)
    idg = jnp.sum(dg[32:], axis=0)
    on = lax.rsqrt(jnp.maximum(od, 1.0))[:, None]
    inn = lax.rsqrt(jnp.maximum(idg, 1.0))[:, None]
    on_ref[...] = on
    in_ref[...] = inn
    xs_ref[...] = x_ref[...] * on


def _prep(deg64, x_pad):
    return pl.pallas_call(
        _prep_body,
        grid=(NP // _RB,),
        in_specs=[
            pl.BlockSpec((64, _RB), lambda i: (0, i)),
            pl.BlockSpec((_RB, D_IN), lambda i: (i, 0)),
        ],
        out_specs=[
            pl.BlockSpec((_RB, D_IN), lambda i: (i, 0)),
            pl.BlockSpec((_RB, 1), lambda i: (i, 0)),
            pl.BlockSpec((_RB, 1), lambda i: (i, 0)),
        ],
        out_shape=[
            jax.ShapeDtypeStruct((NP, D_IN), _f32),
            jax.ShapeDtypeStruct((NP, 1), _f32),
            jax.ShapeDtypeStruct((NP, 1), _f32),
        ],
    )(deg64, x_pad)


_RD = 512


def _mlp_body(p0, p1, on, inn, w1, b1, w2, z0, z1):
    a = (p0[...] + p1[...]) * inn[...]
    h = lax.dot_general(a, w1[...], (((1,), (0,)), ((), ())),
                        precision=lax.Precision.HIGHEST,
                        preferred_element_type=_f32)
    h = jnp.maximum(h + b1[...], 0.0)
    t = h * on[...]
    z = lax.dot_general(t, w2[...], (((1,), (0,)), ((), ())),
                        precision=lax.Precision.HIGHEST,
                        preferred_element_type=_f32)
    z0[...] = z[:, :D_IN]
    z1[...] = z[:, D_IN:]


def _mlp(p0, p1, on, inn, w1, b1r, w2):
    return pl.pallas_call(
        _mlp_body,
        grid=(NP // _RD,),
        in_specs=[
            pl.BlockSpec((_RD, D_IN), lambda i: (i, 0)),
            pl.BlockSpec((_RD, D_IN), lambda i: (i, 0)),
            pl.BlockSpec((_RD, 1), lambda i: (i, 0)),
            pl.BlockSpec((_RD, 1), lambda i: (i, 0)),
            pl.BlockSpec((D_IN, H1), lambda i: (0, 0)),
            pl.BlockSpec((1, H1), lambda i: (0, 0)),
            pl.BlockSpec((H1, H2), lambda i: (0, 0)),
        ],
        out_specs=[
            pl.BlockSpec((_RD, D_IN), lambda i: (i, 0)),
            pl.BlockSpec((_RD, D_IN), lambda i: (i, 0)),
        ],
        out_shape=[
            jax.ShapeDtypeStruct((NP, D_IN), _f32),
            jax.ShapeDtypeStruct((NP, D_IN), _f32),
        ],
    )(p0, p1, on, inn, w1, b1r, w2)


_RF = 2000


def _fin_body(a0, a1, inn, b2, out):
    z = jnp.concatenate([a0[...], a1[...]], axis=1)
    out[...] = z * inn[...] + b2[...]


def _fin(a0, a1, inn, b2r):
    return pl.pallas_call(
        _fin_body,
        grid=(N // _RF,),
        in_specs=[
            pl.BlockSpec((_RF, D_IN), lambda i: (i, 0)),
            pl.BlockSpec((_RF, D_IN), lambda i: (i, 0)),
            pl.BlockSpec((_RF, 1), lambda i: (i, 0)),
            pl.BlockSpec((1, H2), lambda i: (0, 0)),
        ],
        out_specs=pl.BlockSpec((_RF, H2), lambda i: (i, 0)),
        out_shape=jax.ShapeDtypeStruct((N, H2), _f32),
    )(a0, a1, inn, b2r)


def kernel(in_feat, edge_index, W1, b1, W2, b2):
    src = edge_index[0].astype(jnp.int32)
    dst = edge_index[1].astype(jnp.int32)
    pad = jnp.full((EP - E,), N, jnp.int32)
    srcp = jnp.concatenate([src, pad])
    dstp = jnp.concatenate([dst, pad])
    x_pad = jnp.pad(in_feat, ((0, NP - N), (0, 0)))
    b1r = b1.reshape(1, H1)
    b2r = b2.reshape(1, H2)

    deg_flat = _deg_kernel(srcp, dstp)
    deg64 = deg_flat.reshape(64, NP)
    xs, on, inn = _prep(deg64, x_pad)
    p0, p1 = _mp1_kernel(xs, srcp, dstp)
    z0, z1 = _mlp(p0, p1, on, inn, W1, b1r, W2)
    a0, a1 = _mp2_kernel(z0, z1, srcp, dstp)
    return _fin(a0, a1, inn, b2r)


# speedup vs baseline: 5.1310x; 1.0574x over previous
import functools

import jax
import jax.numpy as jnp
from jax import lax
from jax.experimental import pallas as pl
from jax.experimental.pallas import tpu as pltpu
from jax.experimental.pallas import tpu_sc as plsc

N = 10000
NP = 10240
D_IN = 128
H1 = 512
H2 = 256
E = 320000
B = 128
NB1 = 80
EP = 32 * NB1 * B
EPT1 = EP // 32
NB2 = 2 * NB1
EPT2 = EP // 16
RPT = NP // 16

_mesh = plsc.VectorSubcoreMesh(core_axis_name="c", subcore_axis_name="s")
_f32 = jnp.float32
_sc_params = pltpu.CompilerParams(needs_layout_passes=False)


def _zero_vmem_block(zb, rows):
    z16 = jnp.zeros((16,), _f32)

    def body(i, carry):
        r = i // 8
        k = i % 8
        zb[r, pl.ds(k * 16, 16)] = z16
        return carry

    lax.fori_loop(0, rows * 8, body, 0)


def _zero_spmem_rows(agg, zb, row0):
    def zcp(t, carry):
        pltpu.sync_copy(zb, agg.at[pl.ds(row0 + t * 64, 64)])
        return carry

    lax.fori_loop(0, RPT // 64, zcp, 0)


def _edge_pipeline(table, agg, idx_src, idx_bufs, idx_sems, rows2,
                   gsems, ssems, nb):
    src3d, dst3d, ibase = idx_src
    ia, ja = idx_bufs
    ias, jas = idx_sems

    def start_ia(k, g):
        pltpu.async_copy(src3d.at[pl.ds(ibase + g, 1)], ia[k], ias[k])

    def wait_ia(k):
        pltpu.make_async_copy(src3d.at[pl.ds(ibase, 1)], ia[k], ias[k]).wait()

    def start_ja(k, g):
        pltpu.async_copy(dst3d.at[pl.ds(ibase + g, 1)], ja[k], jas[k])

    def wait_ja(k):
        pltpu.make_async_copy(dst3d.at[pl.ds(ibase, 1)], ja[k], jas[k]).wait()

    def start_g(k):
        pltpu.async_copy(table.at[ia[k].at[0, 0]], rows2[k], gsems[k])

    def wait_g(k):
        pltpu.make_async_copy(table.at[ia[0].at[0, 0]], rows2[k],
                              gsems[k]).wait()

    def start_s(k):
        pltpu.async_copy(rows2[k], agg.at[ja[k].at[0, 0]], ssems[k], add=True)

    def wait_s(k):
        pltpu.make_async_copy(rows2[k], agg.at[ja[0].at[0, 0]],
                              ssems[k]).wait()

    def step(g, k, first):
        k2 = 1 - k
        wait_g(k)
        wait_ja(k)
        start_s(k)
        start_ia(k, jnp.minimum(g + 2, nb - 1))
        if not first:
            wait_s(k2)
        start_ja(k2, g + 1)
        wait_ia(k2)
        start_g(k2)

    start_ia(0, 0)
    start_ja(0, 0)
    start_ia(1, 1)
    wait_ia(0)
    start_g(0)
    step(0, 0, True)

    def body(t, carry):
        g = 1 + 2 * t
        step(g, 1, False)
        step(g + 1, 0, False)
        return carry

    lax.fori_loop(0, (nb - 2) // 2, body, 0)
    wait_g(1)
    wait_ja(1)
    start_s(1)
    wait_s(0)
    wait_s(1)
    wait_ia(0)


@functools.partial(
    pl.kernel,
    mesh=_mesh,
    out_type=jax.ShapeDtypeStruct((64 * NP,), _f32),
    scratch_types=[
        pltpu.VMEM((NP,), _f32),
        pltpu.VMEM((NP,), _f32),
        pltpu.VMEM((EPT1,), jnp.int32),
        pltpu.VMEM((EPT1,), jnp.int32),
    ],
    compiler_params=_sc_params,
)
def _deg_kernel(srcp, dstp, out, hs, hd, ib, jb):
    c = lax.axis_index("c")
    s = lax.axis_index("s")
    wid = s * 2 + c
    z16 = jnp.zeros((16,), _f32)

    def zero(i, carry):
        hs[pl.ds(i * 16, 16)] = z16
        hd[pl.ds(i * 16, 16)] = z16
        return carry

    lax.fori_loop(0, NP // 16, zero, 0)

    pltpu.sync_copy(srcp.at[pl.ds(wid * EPT1, EPT1)], ib)
    pltpu.sync_copy(dstp.at[pl.ds(wid * EPT1, EPT1)], jb)
    ones = jnp.ones((16,), _f32)

    def batch(q, carry):
        plsc.addupdate_scatter(hs, [ib[pl.ds(q * 16, 16)]], ones)
        plsc.addupdate_scatter(hd, [jb[pl.ds(q * 16, 16)]], ones)
        return carry

    lax.fori_loop(0, EPT1 // 16, batch, 0)
    pltpu.sync_copy(hs, out.at[pl.ds(wid * NP, NP)])
    pltpu.sync_copy(hd, out.at[pl.ds((32 + wid) * NP, NP)])


_mp_scratch = [
    pltpu.VMEM_SHARED((NP, D_IN), _f32),
    pltpu.VMEM((1, 1, B), jnp.int32),
    pltpu.VMEM((1, 1, B), jnp.int32),
    pltpu.VMEM((1, 1, B), jnp.int32),
    pltpu.VMEM((1, 1, B), jnp.int32),
    pltpu.VMEM((B, D_IN), _f32),
    pltpu.VMEM((B, D_IN), _f32),
    pltpu.VMEM((64, D_IN), _f32),
] + [pltpu.SemaphoreType.DMA] * 8


@functools.partial(
    pl.kernel,
    mesh=_mesh,
    out_type=[
        jax.ShapeDtypeStruct((NP, D_IN), _f32),
        jax.ShapeDtypeStruct((NP, D_IN), _f32),
    ],
    scratch_types=_mp_scratch,
    compiler_params=_sc_params,
)
def _mp1_kernel(xs, src3d, dst3d, out0, out1, agg, ia0, ia1, ja0, ja1,
                r0, r1, zb, g0, g1, s0, s1, x0, x1, y0, y1):
    c = lax.axis_index("c")
    s = lax.axis_index("s")
    wid = s * 2 + c
    row0 = s * RPT

    _zero_vmem_block(zb, 64)
    _zero_spmem_rows(agg, zb, row0)
    plsc.subcore_barrier()

    _edge_pipeline(xs, agg, (src3d, dst3d, wid * NB1),
                   ((ia0, ia1), (ja0, ja1)), ((x0, x1), (y0, y1)),
                   (r0, r1), (g0, g1), (s0, s1), NB1)
    plsc.subcore_barrier()

    @pl.when(c == 0)
    def _():
        pltpu.sync_copy(agg.at[pl.ds(row0, RPT)], out0.at[pl.ds(row0, RPT)])

    @pl.when(c == 1)
    def _():
        pltpu.sync_copy(agg.at[pl.ds(row0, RPT)], out1.at[pl.ds(row0, RPT)])


@functools.partial(
    pl.kernel,
    mesh=_mesh,
    out_type=[
        jax.ShapeDtypeStruct((NP, D_IN), _f32),
        jax.ShapeDtypeStruct((NP, D_IN), _f32),
    ],
    scratch_types=_mp_scratch,
    compiler_params=_sc_params,
)
def _mp2_kernel(z0, z1, src3d, dst3d, out0, out1, agg, ia0, ia1, ja0, ja1,
                r0, r1, zb, g0, g1, s0, s1, x0, x1, y0, y1):
    c = lax.axis_index("c")
    s = lax.axis_index("s")
    row0 = s * RPT

    _zero_vmem_block(zb, 64)
    _zero_spmem_rows(agg, zb, row0)
    plsc.subcore_barrier()

    def run(table, out):
        _edge_pipeline(table, agg, (src3d, dst3d, s * NB2),
                       ((ia0, ia1), (ja0, ja1)), ((x0, x1), (y0, y1)),
                       (r0, r1), (g0, g1), (s0, s1), NB2)
        plsc.subcore_barrier()
        pltpu.sync_copy(agg.at[pl.ds(row0, RPT)], out.at[pl.ds(row0, RPT)])

    @pl.when(c == 0)
    def _():
        run(z0, out0)

    @pl.when(c == 1)
    def _():
        run(z1, out1)


_RB = 1024


def _prep_body(deg_ref, x_ref, xs_ref, on_ref, in_ref):
    dg = deg_ref[...]
    od = jnp.sum(dg[:32], axis=0)
    idg = jnp.sum(dg[32:], axis=0)
    on = lax.rsqrt(jnp.maximum(od, 1.0))[:, None]
    inn = lax.rsqrt(jnp.maximum(idg, 1.0))[:, None]
    on_ref[...] = on
    in_ref[...] = inn
    xs_ref[...] = x_ref[...] * on


def _prep(deg64, x_pad):
    return pl.pallas_call(
        _prep_body,
        grid=(NP // _RB,),
        in_specs=[
            pl.BlockSpec((64, _RB), lambda i: (0, i)),
            pl.BlockSpec((_RB, D_IN), lambda i: (i, 0)),
        ],
        out_specs=[
            pl.BlockSpec((_RB, D_IN), lambda i: (i, 0)),
            pl.BlockSpec((_RB, 1), lambda i: (i, 0)),
            pl.BlockSpec((_RB, 1), lambda i: (i, 0)),
        ],
        out_shape=[
            jax.ShapeDtypeStruct((NP, D_IN), _f32),
            jax.ShapeDtypeStruct((NP, 1), _f32),
            jax.ShapeDtypeStruct((NP, 1), _f32),
        ],
    )(deg64, x_pad)


_RD = 512


def _mlp_body(p0, p1, on, inn, w1, b1, w2, z0, z1):
    a = (p0[...] + p1[...]) * inn[...]
    h = lax.dot_general(a, w1[...], (((1,), (0,)), ((), ())),
                        precision=lax.Precision.HIGHEST,
                        preferred_element_type=_f32)
    h = jnp.maximum(h + b1[...], 0.0)
    t = h * on[...]
    z = lax.dot_general(t, w2[...], (((1,), (0,)), ((), ())),
                        precision=lax.Precision.HIGHEST,
                        preferred_element_type=_f32)
    z0[...] = z[:, :D_IN]
    z1[...] = z[:, D_IN:]


def _mlp(p0, p1, on, inn, w1, b1r, w2):
    return pl.pallas_call(
        _mlp_body,
        grid=(NP // _RD,),
        in_specs=[
            pl.BlockSpec((_RD, D_IN), lambda i: (i, 0)),
            pl.BlockSpec((_RD, D_IN), lambda i: (i, 0)),
            pl.BlockSpec((_RD, 1), lambda i: (i, 0)),
            pl.BlockSpec((_RD, 1), lambda i: (i, 0)),
            pl.BlockSpec((D_IN, H1), lambda i: (0, 0)),
            pl.BlockSpec((1, H1), lambda i: (0, 0)),
            pl.BlockSpec((H1, H2), lambda i: (0, 0)),
        ],
        out_specs=[
            pl.BlockSpec((_RD, D_IN), lambda i: (i, 0)),
            pl.BlockSpec((_RD, D_IN), lambda i: (i, 0)),
        ],
        out_shape=[
            jax.ShapeDtypeStruct((NP, D_IN), _f32),
            jax.ShapeDtypeStruct((NP, D_IN), _f32),
        ],
    )(p0, p1, on, inn, w1, b1r, w2)


_RF = 2000


def _fin_body(a0, a1, inn, b2, out):
    z = jnp.concatenate([a0[...], a1[...]], axis=1)
    out[...] = z * inn[...] + b2[...]


def _fin(a0, a1, inn, b2r):
    return pl.pallas_call(
        _fin_body,
        grid=(N // _RF,),
        in_specs=[
            pl.BlockSpec((_RF, D_IN), lambda i: (i, 0)),
            pl.BlockSpec((_RF, D_IN), lambda i: (i, 0)),
            pl.BlockSpec((_RF, 1), lambda i: (i, 0)),
            pl.BlockSpec((1, H2), lambda i: (0, 0)),
        ],
        out_specs=pl.BlockSpec((_RF, H2), lambda i: (i, 0)),
        out_shape=jax.ShapeDtypeStruct((N, H2), _f32),
    )(a0, a1, inn, b2r)


def kernel(in_feat, edge_index, W1, b1, W2, b2):
    src = edge_index[0].astype(jnp.int32)
    dst = edge_index[1].astype(jnp.int32)
    pad = jnp.full((EP - E,), N, jnp.int32)
    srcp = jnp.concatenate([src, pad])
    dstp = jnp.concatenate([dst, pad])
    src3d = srcp.reshape(EP // B, 1, B)
    dst3d = dstp.reshape(EP // B, 1, B)
    x_pad = jnp.pad(in_feat, ((0, NP - N), (0, 0)))
    b1r = b1.reshape(1, H1)
    b2r = b2.reshape(1, H2)

    deg_flat = _deg_kernel(srcp, dstp)
    deg64 = deg_flat.reshape(64, NP)
    xs, on, inn = _prep(deg64, x_pad)
    p0, p1 = _mp1_kernel(xs, src3d, dst3d)
    z0, z1 = _mlp(p0, p1, on, inn, W1, b1r, W2)
    a0, a1 = _mp2_kernel(z0, z1, src3d, dst3d)
    return _fin(a0, a1, inn, b2r)


# speedup vs baseline: 11.8292x; 2.3054x over previous
import functools

import jax
import jax.numpy as jnp
from jax import lax
from jax.experimental import pallas as pl
from jax.experimental.pallas import tpu as pltpu
from jax.experimental.pallas import tpu_sc as plsc

N = 10000
NP = 10240
D_IN = 128
H1 = 512
H2 = 256
E = 320000
B = 128
NB1 = 80
EP = 32 * NB1 * B
EPT1 = EP // 32
NB2 = 2 * NB1
EPT2 = EP // 16
RPT = NP // 16

_mesh = plsc.VectorSubcoreMesh(core_axis_name="c", subcore_axis_name="s")
_f32 = jnp.float32
_sc_params = pltpu.CompilerParams(needs_layout_passes=False)


def _zero_vmem_block(zb, rows):
    z16 = jnp.zeros((16,), _f32)

    def body(i, carry):
        r = i // 8
        k = i % 8
        zb[r, pl.ds(k * 16, 16)] = z16
        return carry

    lax.fori_loop(0, rows * 8, body, 0)


def _zero_spmem_rows(agg, zb, row0):
    def zcp(t, carry):
        pltpu.sync_copy(zb, agg.at[pl.ds(row0 + t * 64, 64)])
        return carry

    lax.fori_loop(0, RPT // 64, zcp, 0)


def _edge_pipeline(table, agg, idx_src, idx_bufs, idx_sems, rows2,
                   gsems, ssems, nb):
    src3d, dst3d, ibase = idx_src
    ia, ja = idx_bufs
    ias, jas = idx_sems

    def start_ia(k, g):
        pltpu.async_copy(src3d.at[pl.ds(ibase + g, 1)], ia[k], ias[k])

    def wait_ia(k):
        pltpu.make_async_copy(src3d.at[pl.ds(ibase, 1)], ia[k], ias[k]).wait()

    def start_ja(k, g):
        pltpu.async_copy(dst3d.at[pl.ds(ibase + g, 1)], ja[k], jas[k])

    def wait_ja(k):
        pltpu.make_async_copy(dst3d.at[pl.ds(ibase, 1)], ja[k], jas[k]).wait()

    def start_g(k):
        pltpu.async_copy(table.at[ia[k].at[0, 0]], rows2[k], gsems[k])

    def wait_g(k):
        pltpu.make_async_copy(table.at[ia[0].at[0, 0]], rows2[k],
                              gsems[k]).wait()

    def start_s(k):
        pltpu.async_copy(rows2[k], agg.at[ja[k].at[0, 0]], ssems[k], add=True)

    def wait_s(k):
        pltpu.make_async_copy(rows2[k], agg.at[ja[0].at[0, 0]],
                              ssems[k]).wait()

    def step(g, k, first):
        k2 = 1 - k
        wait_g(k)
        wait_ja(k)
        start_s(k)
        start_ia(k, jnp.minimum(g + 2, nb - 1))
        if not first:
            wait_s(k2)
        start_ja(k2, g + 1)
        wait_ia(k2)
        start_g(k2)

    start_ia(0, 0)
    start_ja(0, 0)
    start_ia(1, 1)
    wait_ia(0)
    start_g(0)
    step(0, 0, True)

    def body(t, carry):
        g = 1 + 2 * t
        step(g, 1, False)
        step(g + 1, 0, False)
        return carry

    lax.fori_loop(0, (nb - 2) // 2, body, 0)
    wait_g(1)
    wait_ja(1)
    start_s(1)
    wait_s(0)
    wait_s(1)
    wait_ia(0)


@functools.partial(
    pl.kernel,
    mesh=_mesh,
    out_type=jax.ShapeDtypeStruct((64 * NP,), _f32),
    scratch_types=[
        pltpu.VMEM((NP,), _f32),
        pltpu.VMEM((NP,), _f32),
        pltpu.VMEM((EPT1,), jnp.int32),
        pltpu.VMEM((EPT1,), jnp.int32),
    ],
    compiler_params=_sc_params,
)
def _deg_kernel(srcp, dstp, out, hs, hd, ib, jb):
    c = lax.axis_index("c")
    s = lax.axis_index("s")
    wid = s * 2 + c
    z16 = jnp.zeros((16,), _f32)

    def zero(i, carry):
        hs[pl.ds(i * 16, 16)] = z16
        hd[pl.ds(i * 16, 16)] = z16
        return carry

    lax.fori_loop(0, NP // 16, zero, 0)

    pltpu.sync_copy(srcp.at[pl.ds(wid * EPT1, EPT1)], ib)
    pltpu.sync_copy(dstp.at[pl.ds(wid * EPT1, EPT1)], jb)
    ones = jnp.ones((16,), _f32)

    def batch(q, carry):
        plsc.addupdate_scatter(hs, [ib[pl.ds(q * 16, 16)]], ones)
        plsc.addupdate_scatter(hd, [jb[pl.ds(q * 16, 16)]], ones)
        return carry

    lax.fori_loop(0, EPT1 // 16, batch, 0)
    pltpu.sync_copy(hs, out.at[pl.ds(wid * NP, NP)])
    pltpu.sync_copy(hd, out.at[pl.ds((32 + wid) * NP, NP)])


_mp_scratch = [
    pltpu.VMEM_SHARED((NP, D_IN), _f32),
    pltpu.VMEM((1, 1, B), jnp.int32),
    pltpu.VMEM((1, 1, B), jnp.int32),
    pltpu.VMEM((1, 1, B), jnp.int32),
    pltpu.VMEM((1, 1, B), jnp.int32),
    pltpu.VMEM((B, D_IN), _f32),
    pltpu.VMEM((B, D_IN), _f32),
    pltpu.VMEM((64, D_IN), _f32),
] + [pltpu.SemaphoreType.DMA] * 8


@functools.partial(
    pl.kernel,
    mesh=_mesh,
    out_type=[
        jax.ShapeDtypeStruct((NP, D_IN), _f32),
        jax.ShapeDtypeStruct((NP, D_IN), _f32),
    ],
    scratch_types=_mp_scratch,
    compiler_params=_sc_params,
)
def _mp1_kernel(xs, src3d, dst3d, out0, out1, agg, ia0, ia1, ja0, ja1,
                r0, r1, zb, g0, g1, s0, s1, x0, x1, y0, y1):
    c = lax.axis_index("c")
    s = lax.axis_index("s")
    wid = s * 2 + c
    row0 = s * RPT

    _zero_vmem_block(zb, 64)
    _zero_spmem_rows(agg, zb, row0)
    plsc.subcore_barrier()

    _edge_pipeline(xs, agg, (src3d, dst3d, wid * NB1),
                   ((ia0, ia1), (ja0, ja1)), ((x0, x1), (y0, y1)),
                   (r0, r1), (g0, g1), (s0, s1), NB1)
    plsc.subcore_barrier()

    @pl.when(c == 0)
    def _():
        pltpu.sync_copy(agg.at[pl.ds(row0, RPT)], out0.at[pl.ds(row0, RPT)])

    @pl.when(c == 1)
    def _():
        pltpu.sync_copy(agg.at[pl.ds(row0, RPT)], out1.at[pl.ds(row0, RPT)])


@functools.partial(
    pl.kernel,
    mesh=_mesh,
    out_type=[
        jax.ShapeDtypeStruct((NP, D_IN), _f32),
        jax.ShapeDtypeStruct((NP, D_IN), _f32),
    ],
    scratch_types=_mp_scratch,
    compiler_params=_sc_params,
)
def _mp2_kernel(z0, z1, src3d, dst3d, out0, out1, agg, ia0, ia1, ja0, ja1,
                r0, r1, zb, g0, g1, s0, s1, x0, x1, y0, y1):
    c = lax.axis_index("c")
    s = lax.axis_index("s")
    row0 = s * RPT

    _zero_vmem_block(zb, 64)
    _zero_spmem_rows(agg, zb, row0)
    plsc.subcore_barrier()

    def run(table, out):
        _edge_pipeline(table, agg, (src3d, dst3d, s * NB2),
                       ((ia0, ia1), (ja0, ja1)), ((x0, x1), (y0, y1)),
                       (r0, r1), (g0, g1), (s0, s1), NB2)
        plsc.subcore_barrier()
        pltpu.sync_copy(agg.at[pl.ds(row0, RPT)], out.at[pl.ds(row0, RPT)])

    @pl.when(c == 0)
    def _():
        run(z0, out0)

    @pl.when(c == 1)
    def _():
        run(z1, out1)


_RB = 1024


def _prep_body(deg_ref, x_ref, xs_ref, on_ref, in_ref):
    dg = deg_ref[...]
    od = jnp.sum(dg[:32], axis=0)
    idg = jnp.sum(dg[32:], axis=0)
    on = lax.rsqrt(jnp.maximum(od, 1.0))[:, None]
    inn = lax.rsqrt(jnp.maximum(idg, 1.0))[:, None]
    on_ref[...] = on
    in_ref[...] = inn
    xs_ref[...] = x_ref[...] * on


def _prep(deg64, x_pad):
    return pl.pallas_call(
        _prep_body,
        grid=(NP // _RB,),
        in_specs=[
            pl.BlockSpec((64, _RB), lambda i: (0, i)),
            pl.BlockSpec((_RB, D_IN), lambda i: (i, 0)),
        ],
        out_specs=[
            pl.BlockSpec((_RB, D_IN), lambda i: (i, 0)),
            pl.BlockSpec((_RB, 1), lambda i: (i, 0)),
            pl.BlockSpec((_RB, 1), lambda i: (i, 0)),
        ],
        out_shape=[
            jax.ShapeDtypeStruct((NP, D_IN), _f32),
            jax.ShapeDtypeStruct((NP, 1), _f32),
            jax.ShapeDtypeStruct((NP, 1), _f32),
        ],
    )(deg64, x_pad)


_RD = 512


def _mlp_body(p0, p1, on, inn, w1, b1, w2, z0, z1):
    a = (p0[...] + p1[...]) * inn[...]
    h = lax.dot_general(a, w1[...], (((1,), (0,)), ((), ())),
                        precision=lax.Precision.HIGHEST,
                        preferred_element_type=_f32)
    h = jnp.maximum(h + b1[...], 0.0)
    t = h * on[...]
    z = lax.dot_general(t, w2[...], (((1,), (0,)), ((), ())),
                        precision=lax.Precision.HIGHEST,
                        preferred_element_type=_f32)
    z0[...] = z[:, :D_IN]
    z1[...] = z[:, D_IN:]


def _mlp(p0, p1, on, inn, w1, b1r, w2):
    return pl.pallas_call(
        _mlp_body,
        grid=(NP // _RD,),
        in_specs=[
            pl.BlockSpec((_RD, D_IN), lambda i: (i, 0)),
            pl.BlockSpec((_RD, D_IN), lambda i: (i, 0)),
            pl.BlockSpec((_RD, 1), lambda i: (i, 0)),
            pl.BlockSpec((_RD, 1), lambda i: (i, 0)),
            pl.BlockSpec((D_IN, H1), lambda i: (0, 0)),
            pl.BlockSpec((1, H1), lambda i: (0, 0)),
            pl.BlockSpec((H1, H2), lambda i: (0, 0)),
        ],
        out_specs=[
            pl.BlockSpec((_RD, D_IN), lambda i: (i, 0)),
            pl.BlockSpec((_RD, D_IN), lambda i: (i, 0)),
        ],
        out_shape=[
            jax.ShapeDtypeStruct((NP, D_IN), _f32),
            jax.ShapeDtypeStruct((NP, D_IN), _f32),
        ],
    )(p0, p1, on, inn, w1, b1r, w2)


_RF = 2000


def _fin_body(a0, a1, inn, b2, out):
    z = jnp.concatenate([a0[...], a1[...]], axis=1)
    out[...] = z * inn[...] + b2[...]


def _fin(a0, a1, inn, b2r):
    return pl.pallas_call(
        _fin_body,
        grid=(N // _RF,),
        in_specs=[
            pl.BlockSpec((_RF, D_IN), lambda i: (i, 0)),
            pl.BlockSpec((_RF, D_IN), lambda i: (i, 0)),
            pl.BlockSpec((_RF, 1), lambda i: (i, 0)),
            pl.BlockSpec((1, H2), lambda i: (0, 0)),
        ],
        out_specs=pl.BlockSpec((_RF, H2), lambda i: (i, 0)),
        out_shape=jax.ShapeDtypeStruct((N, H2), _f32),
    )(a0, a1, inn, b2r)


def kernel(in_feat, edge_index, W1, b1, W2, b2):
    src = edge_index[0].astype(jnp.int32)
    dst = edge_index[1].astype(jnp.int32)
    pad = N + jnp.arange(EP - E, dtype=jnp.int32) % (NP - N)
    srcp = jnp.concatenate([src, pad])
    dstp = jnp.concatenate([dst, pad])
    src3d = srcp.reshape(EP // B, 1, B)
    dst3d = dstp.reshape(EP // B, 1, B)
    x_pad = jnp.pad(in_feat, ((0, NP - N), (0, 0)))
    b1r = b1.reshape(1, H1)
    b2r = b2.reshape(1, H2)

    deg_flat = _deg_kernel(srcp, dstp)
    deg64 = deg_flat.reshape(64, NP)
    xs, on, inn = _prep(deg64, x_pad)
    p0, p1 = _mp1_kernel(xs, src3d, dst3d)
    z0, z1 = _mlp(p0, p1, on, inn, W1, b1r, W2)
    a0, a1 = _mp2_kernel(z0, z1, src3d, dst3d)
    return _fin(a0, a1, inn, b2r)


# speedup vs baseline: 11.9882x; 1.0134x over previous
import functools

import jax
import jax.numpy as jnp
from jax import lax
from jax.experimental import pallas as pl
from jax.experimental.pallas import tpu as pltpu
from jax.experimental.pallas import tpu_sc as plsc

N = 10000
NP = 10240
D_IN = 128
H1 = 512
H2 = 256
E = 320000
B = 96
NB1 = 108
EP = 32 * NB1 * B
EPT1 = EP // 32
NB2 = 2 * NB1
EPT2 = EP // 16
RPT = NP // 16

_mesh = plsc.VectorSubcoreMesh(core_axis_name="c", subcore_axis_name="s")
_f32 = jnp.float32
_sc_params = pltpu.CompilerParams(needs_layout_passes=False)


def _zero_vmem_block(zb, rows):
    z16 = jnp.zeros((16,), _f32)

    def body(i, carry):
        r = i // 8
        k = i % 8
        zb[r, pl.ds(k * 16, 16)] = z16
        return carry

    lax.fori_loop(0, rows * 8, body, 0)


def _zero_spmem_rows(agg, zb, row0):
    def zcp(t, carry):
        pltpu.sync_copy(zb, agg.at[pl.ds(row0 + t * 64, 64)])
        return carry

    lax.fori_loop(0, RPT // 64, zcp, 0)


def _edge_pipeline(table, agg, idx_src, idx_bufs, idx_sems, rows3,
                   gsems, ssems, nb):
    src3d, dst3d, ibase = idx_src
    ia, ja = idx_bufs
    ias, jas = idx_sems

    def start_ia(k, g):
        pltpu.async_copy(src3d.at[pl.ds(ibase + g, 1)], ia[k], ias[k])

    def wait_ia(k):
        pltpu.make_async_copy(src3d.at[pl.ds(ibase, 1)], ia[k], ias[k]).wait()

    def start_ja(k, g):
        pltpu.async_copy(dst3d.at[pl.ds(ibase + g, 1)], ja[k], jas[k])

    def wait_ja(k):
        pltpu.make_async_copy(dst3d.at[pl.ds(ibase, 1)], ja[k], jas[k]).wait()

    def start_g(k):
        pltpu.async_copy(table.at[ia[k].at[0, 0]], rows3[k], gsems[k])

    def wait_g(k):
        pltpu.make_async_copy(table.at[ia[0].at[0, 0]], rows3[k],
                              gsems[k]).wait()

    def start_s(k):
        pltpu.async_copy(rows3[k], agg.at[ja[k].at[0, 0]], ssems[k], add=True)

    def wait_s(k):
        pltpu.make_async_copy(rows3[k], agg.at[ja[0].at[0, 0]],
                              ssems[k]).wait()

    def step(g, k, m, first):
        wait_g(k)
        wait_ja(k)
        start_s(k)
        start_ia(k, jnp.minimum(g + 3, nb - 1))
        if not first:
            wait_s(m)
        start_ja(m, g + 1)
        wait_ia(m)
        start_g(m)

    start_ia(0, 0)
    start_ja(0, 0)
    start_ia(1, 1)
    start_ia(2, 2)
    wait_ia(0)
    start_g(0)
    step(0, 0, 1, True)
    step(1, 1, 2, True)

    def body(t, carry):
        g2 = 2 + 3 * t
        step(g2, 2, 0, False)
        step(g2 + 1, 0, 1, False)
        step(g2 + 2, 1, 2, False)
        return carry

    lax.fori_loop(0, (nb - 3) // 3, body, 0)
    wait_g(2)
    wait_ja(2)
    start_s(2)
    wait_s(0)
    wait_s(1)
    wait_s(2)
    wait_ia(0)
    wait_ia(1)


@functools.partial(
    pl.kernel,
    mesh=_mesh,
    out_type=jax.ShapeDtypeStruct((64 * NP,), _f32),
    scratch_types=[
        pltpu.VMEM((NP,), _f32),
        pltpu.VMEM((NP,), _f32),
        pltpu.VMEM((EPT1,), jnp.int32),
        pltpu.VMEM((EPT1,), jnp.int32),
    ],
    compiler_params=_sc_params,
)
def _deg_kernel(srcp, dstp, out, hs, hd, ib, jb):
    c = lax.axis_index("c")
    s = lax.axis_index("s")
    wid = s * 2 + c
    z16 = jnp.zeros((16,), _f32)

    def zero(i, carry):
        hs[pl.ds(i * 16, 16)] = z16
        hd[pl.ds(i * 16, 16)] = z16
        return carry

    lax.fori_loop(0, NP // 16, zero, 0)

    pltpu.sync_copy(srcp.at[pl.ds(wid * EPT1, EPT1)], ib)
    pltpu.sync_copy(dstp.at[pl.ds(wid * EPT1, EPT1)], jb)
    ones = jnp.ones((16,), _f32)

    def batch(q, carry):
        plsc.addupdate_scatter(hs, [ib[pl.ds(q * 16, 16)]], ones)
        plsc.addupdate_scatter(hd, [jb[pl.ds(q * 16, 16)]], ones)
        return carry

    lax.fori_loop(0, EPT1 // 16, batch, 0)
    pltpu.sync_copy(hs, out.at[pl.ds(wid * NP, NP)])
    pltpu.sync_copy(hd, out.at[pl.ds((32 + wid) * NP, NP)])


_mp_scratch = [
    pltpu.VMEM_SHARED((NP, D_IN), _f32),
    pltpu.VMEM((1, 1, B), jnp.int32),
    pltpu.VMEM((1, 1, B), jnp.int32),
    pltpu.VMEM((1, 1, B), jnp.int32),
    pltpu.VMEM((1, 1, B), jnp.int32),
    pltpu.VMEM((1, 1, B), jnp.int32),
    pltpu.VMEM((1, 1, B), jnp.int32),
    pltpu.VMEM((B, D_IN), _f32),
    pltpu.VMEM((B, D_IN), _f32),
    pltpu.VMEM((B, D_IN), _f32),
    pltpu.VMEM((64, D_IN), _f32),
] + [pltpu.SemaphoreType.DMA] * 12


@functools.partial(
    pl.kernel,
    mesh=_mesh,
    out_type=[
        jax.ShapeDtypeStruct((NP, D_IN), _f32),
        jax.ShapeDtypeStruct((NP, D_IN), _f32),
    ],
    scratch_types=_mp_scratch,
    compiler_params=_sc_params,
)
def _mp1_kernel(xs, src3d, dst3d, out0, out1, agg, ia0, ia1, ia2,
                ja0, ja1, ja2, r0, r1, r2, zb,
                g0, g1, g2, s0, s1, s2, x0, x1, x2, y0, y1, y2):
    c = lax.axis_index("c")
    s = lax.axis_index("s")
    wid = s * 2 + c
    row0 = s * RPT

    _zero_vmem_block(zb, 64)
    _zero_spmem_rows(agg, zb, row0)
    plsc.subcore_barrier()

    _edge_pipeline(xs, agg, (src3d, dst3d, wid * NB1),
                   ((ia0, ia1, ia2), (ja0, ja1, ja2)),
                   ((x0, x1, x2), (y0, y1, y2)),
                   (r0, r1, r2), (g0, g1, g2), (s0, s1, s2), NB1)
    plsc.subcore_barrier()

    @pl.when(c == 0)
    def _():
        pltpu.sync_copy(agg.at[pl.ds(row0, RPT)], out0.at[pl.ds(row0, RPT)])

    @pl.when(c == 1)
    def _():
        pltpu.sync_copy(agg.at[pl.ds(row0, RPT)], out1.at[pl.ds(row0, RPT)])


@functools.partial(
    pl.kernel,
    mesh=_mesh,
    out_type=[
        jax.ShapeDtypeStruct((NP, D_IN), _f32),
        jax.ShapeDtypeStruct((NP, D_IN), _f32),
    ],
    scratch_types=_mp_scratch,
    compiler_params=_sc_params,
)
def _mp2_kernel(z0, z1, src3d, dst3d, out0, out1, agg, ia0, ia1, ia2,
                ja0, ja1, ja2, r0, r1, r2, zb,
                g0, g1, g2, s0, s1, s2, x0, x1, x2, y0, y1, y2):
    c = lax.axis_index("c")
    s = lax.axis_index("s")
    row0 = s * RPT

    _zero_vmem_block(zb, 64)
    _zero_spmem_rows(agg, zb, row0)
    plsc.subcore_barrier()

    def run(table, out):
        _edge_pipeline(table, agg, (src3d, dst3d, s * NB2),
                       ((ia0, ia1, ia2), (ja0, ja1, ja2)),
                       ((x0, x1, x2), (y0, y1, y2)),
                       (r0, r1, r2), (g0, g1, g2), (s0, s1, s2), NB2)
        plsc.subcore_barrier()
        pltpu.sync_copy(agg.at[pl.ds(row0, RPT)], out.at[pl.ds(row0, RPT)])

    @pl.when(c == 0)
    def _():
        run(z0, out0)

    @pl.when(c == 1)
    def _():
        run(z1, out1)


_RB = 1024


def _prep_body(deg_ref, x_ref, xs_ref, on_ref, in_ref):
    dg = deg_ref[...]
    od = jnp.sum(dg[:32], axis=0)
    idg = jnp.sum(dg[32:], axis=0)
    on = lax.rsqrt(jnp.maximum(od, 1.0))[:, None]
    inn = lax.rsqrt(jnp.maximum(idg, 1.0))[:, None]
    on_ref[...] = on
    in_ref[...] = inn
    xs_ref[...] = x_ref[...] * on


def _prep(deg64, x_pad):
    return pl.pallas_call(
        _prep_body,
        grid=(NP // _RB,),
        in_specs=[
            pl.BlockSpec((64, _RB), lambda i: (0, i)),
            pl.BlockSpec((_RB, D_IN), lambda i: (i, 0)),
        ],
        out_specs=[
            pl.BlockSpec((_RB, D_IN), lambda i: (i, 0)),
            pl.BlockSpec((_RB, 1), lambda i: (i, 0)),
            pl.BlockSpec((_RB, 1), lambda i: (i, 0)),
        ],
        out_shape=[
            jax.ShapeDtypeStruct((NP, D_IN), _f32),
            jax.ShapeDtypeStruct((NP, 1), _f32),
            jax.ShapeDtypeStruct((NP, 1), _f32),
        ],
    )(deg64, x_pad)


_RD = 512


def _mlp_body(p0, p1, on, inn, w1, b1, w2, z0, z1):
    a = (p0[...] + p1[...]) * inn[...]
    h = lax.dot_general(a, w1[...], (((1,), (0,)), ((), ())),
                        precision=lax.Precision.HIGHEST,
                        preferred_element_type=_f32)
    h = jnp.maximum(h + b1[...], 0.0)
    t = h * on[...]
    z = lax.dot_general(t, w2[...], (((1,), (0,)), ((), ())),
                        precision=lax.Precision.HIGHEST,
                        preferred_element_type=_f32)
    z0[...] = z[:, :D_IN]
    z1[...] = z[:, D_IN:]


def _mlp(p0, p1, on, inn, w1, b1r, w2):
    return pl.pallas_call(
        _mlp_body,
        grid=(NP // _RD,),
        in_specs=[
            pl.BlockSpec((_RD, D_IN), lambda i: (i, 0)),
            pl.BlockSpec((_RD, D_IN), lambda i: (i, 0)),
            pl.BlockSpec((_RD, 1), lambda i: (i, 0)),
            pl.BlockSpec((_RD, 1), lambda i: (i, 0)),
            pl.BlockSpec((D_IN, H1), lambda i: (0, 0)),
            pl.BlockSpec((1, H1), lambda i: (0, 0)),
            pl.BlockSpec((H1, H2), lambda i: (0, 0)),
        ],
        out_specs=[
            pl.BlockSpec((_RD, D_IN), lambda i: (i, 0)),
            pl.BlockSpec((_RD, D_IN), lambda i: (i, 0)),
        ],
        out_shape=[
            jax.ShapeDtypeStruct((NP, D_IN), _f32),
            jax.ShapeDtypeStruct((NP, D_IN), _f32),
        ],
    )(p0, p1, on, inn, w1, b1r, w2)


_RF = 2000


def _fin_body(a0, a1, inn, b2, out):
    z = jnp.concatenate([a0[...], a1[...]], axis=1)
    out[...] = z * inn[...] + b2[...]


def _fin(a0, a1, inn, b2r):
    return pl.pallas_call(
        _fin_body,
        grid=(N // _RF,),
        in_specs=[
            pl.BlockSpec((_RF, D_IN), lambda i: (i, 0)),
            pl.BlockSpec((_RF, D_IN), lambda i: (i, 0)),
            pl.BlockSpec((_RF, 1), lambda i: (i, 0)),
            pl.BlockSpec((1, H2), lambda i: (0, 0)),
        ],
        out_specs=pl.BlockSpec((_RF, H2), lambda i: (i, 0)),
        out_shape=jax.ShapeDtypeStruct((N, H2), _f32),
    )(a0, a1, inn, b2r)


def kernel(in_feat, edge_index, W1, b1, W2, b2):
    src = edge_index[0].astype(jnp.int32)
    dst = edge_index[1].astype(jnp.int32)
    pad = N + jnp.arange(EP - E, dtype=jnp.int32) % (NP - N)
    srcp = jnp.concatenate([src, pad])
    dstp = jnp.concatenate([dst, pad])
    src3d = srcp.reshape(EP // B, 1, B)
    dst3d = dstp.reshape(EP // B, 1, B)
    x_pad = jnp.pad(in_feat, ((0, NP - N), (0, 0)))
    b1r = b1.reshape(1, H1)
    b2r = b2.reshape(1, H2)

    deg_flat = _deg_kernel(srcp, dstp)
    deg64 = deg_flat.reshape(64, NP)
    xs, on, inn = _prep(deg64, x_pad)
    p0, p1 = _mp1_kernel(xs, src3d, dst3d)
    z0, z1 = _mlp(p0, p1, on, inn, W1, b1r, W2)
    a0, a1 = _mp2_kernel(z0, z1, src3d, dst3d)
    return _fin(a0, a1, inn, b2r)
